# Initial kernel scaffold; baseline (speedup 1.0000x reference)
import jax
import jax.numpy as jnp
from jax.experimental import pallas as pl


def kernel(point_seq):
    raise NotImplementedError("write your pallas kernel here")


# speedup vs baseline: 1.3334x; 1.3334x over previous
import functools

import jax
import jax.numpy as jnp
from jax import lax
from jax.experimental import pallas as pl
from jax.experimental.pallas import tpu as pltpu

K = 16
NEG = None


def _tc_knn_body(pts_ref, ptsn_ref, dist_ref, idx_ref, xout_ref, xq_ref):
    t = pl.program_id(1)

    @pl.when(t == 0)
    def _():
        xq_ref[...] = pts_ref[0, 0]

    xkt = ptsn_ref[0, 0]
    xq = xq_ref[...]
    n = xq.shape[1]
    kk = jnp.sum(xkt * xkt, axis=1, keepdims=True)
    qq = jnp.sum(xq * xq, axis=0)
    xkb = xkt.astype(jnp.bfloat16).astype(jnp.float32)
    xqb = xq.astype(jnp.bfloat16).astype(jnp.float32)
    dot = (xkb[:, 0:1] * xqb[0][None, :]
           + xkb[:, 1:2] * xqb[1][None, :]
           + xkb[:, 2:3] * xqb[2][None, :])
    d = (qq[None, :] + kk) - 2.0 * dot

    iota_k = lax.broadcasted_iota(jnp.int32, (n, n), 0)
    inf = jnp.float32(jnp.inf)
    zero = jnp.float32(0.0)
    for j in range(K):
        m = jnp.min(d, axis=0)
        cand = jnp.where(d == m[None, :], iota_k, n)
        aidx = jnp.min(cand, axis=0)
        first = cand == aidx[None, :]
        dist_ref[0, 0, j, :] = m
        idx_ref[0, 0, j, :] = aidx
        if j == 0:
            for c in range(3):
                xnew_c = jnp.sum(jnp.where(first, xkt[:, c:c + 1], zero), axis=0)
                xq_ref[c, :] = xnew_c
                xout_ref[0, 0, c, :] = xnew_c
        if j < K - 1:
            d = jnp.where(first, inf, d)


def _tc_knn(pts8, ptsn8):
    b, t, c8, n = pts8.shape
    grid = (b, t)
    return pl.pallas_call(
        _tc_knn_body,
        grid=grid,
        in_specs=[
            pl.BlockSpec((1, 1, c8, n), lambda bi, ti: (bi, jnp.maximum(ti - 1, 0), 0, 0)),
            pl.BlockSpec((1, 1, n, c8), lambda bi, ti: (bi, jnp.maximum(ti - 1, 0), 0, 0)),
        ],
        out_specs=[
            pl.BlockSpec((1, 1, K, n), lambda bi, ti: (bi, ti, 0, 0)),
            pl.BlockSpec((1, 1, K, n), lambda bi, ti: (bi, ti, 0, 0)),
            pl.BlockSpec((1, 1, c8, n), lambda bi, ti: (bi, ti, 0, 0)),
        ],
        out_shape=[
            jax.ShapeDtypeStruct((b, t, K, n), jnp.float32),
            jax.ShapeDtypeStruct((b, t, K, n), jnp.int32),
            jax.ShapeDtypeStruct((b, t, c8, n), jnp.float32),
        ],
        scratch_shapes=[pltpu.VMEM((c8, n), jnp.float32)],
        compiler_params=pltpu.CompilerParams(
            dimension_semantics=("arbitrary", "arbitrary"),
        ),
    )(pts8, ptsn8)


def kernel(point_seq):
    b, t, n, d = point_seq.shape
    ptsT = jnp.transpose(point_seq, (0, 1, 3, 2))
    pts8 = jnp.concatenate(
        [ptsT, jnp.zeros((b, t, 8 - d, n), jnp.float32)], axis=2)
    ptsn8 = jnp.concatenate(
        [point_seq, jnp.zeros((b, t, n, 8 - d), jnp.float32)], axis=3)

    dist_kn, idx_kn, xout8 = _tc_knn(pts8, ptsn8)

    x_out = jnp.transpose(xout8[:, :, :d, :], (0, 1, 3, 2))

    idxs = jnp.transpose(idx_kn, (0, 1, 3, 2))
    distances = jnp.transpose(dist_kn, (0, 1, 3, 2))
    x2 = jnp.concatenate([point_seq[:, :1], point_seq], axis=1)[:, :-1]
    gathered = jax.vmap(jax.vmap(lambda p, i: p[i]))(x2, idxs)
    anchor = x_out[:, 0][:, None, :, None, :]
    normalized = gathered - anchor
    patchlet_feats = jnp.concatenate([gathered, normalized], axis=-1)
    return patchlet_feats, gathered, distances, idxs, x_out
